# Initial kernel scaffold; baseline (speedup 1.0000x reference)
import jax
import jax.numpy as jnp
from jax.experimental import pallas as pl


def kernel(x, edge_index, W1, b1, W2, b2):
    raise NotImplementedError("write your pallas kernel here")


# speedup vs baseline: 45.5683x; 45.5683x over previous
import functools

import jax
import jax.numpy as jnp
from jax import lax
from jax.experimental import pallas as pl
from jax.experimental.pallas import tpu as pltpu
from jax.experimental.pallas import tpu_sc as plsc

N = 10000
E = 320000
D_IN = 128
D_HID = 16
N_CLASSES = 40

NC, NS, L = 2, 16, 16
NW = NC * NS
EPT = E // NW
CH = 80
NCH = EPT // CH

NP = 10240
RPT = NP // NS

_ROW_BLK = 1280
_N_BLK = NP // _ROW_BLK

_MESH = plsc.VectorSubcoreMesh(core_axis_name="c", subcore_axis_name="s")
_SC_PARAMS = pltpu.CompilerParams(use_tc_tiling_on_sc=False)


@functools.partial(
    pl.kernel,
    out_type=jax.ShapeDtypeStruct((NC, NP, L), jnp.float32),
    mesh=_MESH,
    scratch_types=[
        pltpu.VMEM((NCH, CH), jnp.int32),
        pltpu.VMEM((CH, L), jnp.float32),
        pltpu.VMEM((RPT, L), jnp.float32),
        pltpu.VMEM_SHARED((NP, L), jnp.float32),
    ],
    compiler_params=_SC_PARAMS,
)
def _deg_kernel(dst_hbm, out_hbm, dst_v, ones_v, zero_v, acc_sh):
    cid = lax.axis_index("c")
    sid = lax.axis_index("s")
    wid = cid * NS + sid
    pltpu.sync_copy(dst_hbm.at[wid], dst_v)

    def fill_ones(i, _):
        ones_v[i, :] = jnp.ones((L,), jnp.float32)
        return 0
    lax.fori_loop(0, CH, fill_ones, 0)

    def fill_zero(i, _):
        zero_v[i, :] = jnp.zeros((L,), jnp.float32)
        return 0
    lax.fori_loop(0, RPT, fill_zero, 0)
    pltpu.sync_copy(zero_v, acc_sh.at[pl.ds(sid * RPT, RPT)])
    plsc.subcore_barrier()

    def step(j, _):
        pltpu.sync_copy(ones_v, acc_sh.at[dst_v.at[j]], add=True)
        return 0
    lax.fori_loop(0, NCH, step, 0)
    plsc.subcore_barrier()
    pltpu.sync_copy(acc_sh.at[pl.ds(sid * RPT, RPT)],
                    out_hbm.at[cid, pl.ds(sid * RPT, RPT)])


@functools.partial(
    pl.kernel,
    out_type=jax.ShapeDtypeStruct((NC, NP, L), jnp.float32),
    mesh=_MESH,
    scratch_types=[
        pltpu.VMEM((NCH, CH), jnp.int32),
        pltpu.VMEM((NCH, CH), jnp.int32),
        pltpu.VMEM((2, CH, L), jnp.float32),
        pltpu.VMEM((RPT, L), jnp.float32),
        pltpu.VMEM_SHARED((NP, L), jnp.float32),
        pltpu.VMEM_SHARED((NP, L), jnp.float32),
        pltpu.SemaphoreType.DMA,
        pltpu.SemaphoreType.DMA,
    ],
    compiler_params=_SC_PARAMS,
)
def _agg_kernel(q_hbm, src_hbm, dst_hbm, out_hbm,
                src_v, dst_v, buf, zero_v, q_sh, s_sh, gsem, ssem):
    cid = lax.axis_index("c")
    sid = lax.axis_index("s")
    wid = cid * NS + sid
    pltpu.sync_copy(src_hbm.at[wid], src_v)
    pltpu.sync_copy(dst_hbm.at[wid], dst_v)
    pltpu.sync_copy(q_hbm.at[pl.ds(sid * RPT, RPT)],
                    q_sh.at[pl.ds(sid * RPT, RPT)])

    def fill_zero(i, _):
        zero_v[i, :] = jnp.zeros((L,), jnp.float32)
        return 0
    lax.fori_loop(0, RPT, fill_zero, 0)
    pltpu.sync_copy(zero_v, s_sh.at[pl.ds(sid * RPT, RPT)])
    plsc.subcore_barrier()

    def step(j, _):
        pltpu.sync_copy(q_sh.at[src_v.at[j]], buf.at[0])
        pltpu.sync_copy(buf.at[0], s_sh.at[dst_v.at[j]], add=True)
        return 0
    lax.fori_loop(0, NCH, step, 0)
    plsc.subcore_barrier()
    pltpu.sync_copy(s_sh.at[pl.ds(sid * RPT, RPT)],
                    out_hbm.at[cid, pl.ds(sid * RPT, RPT)])


def _tc1_body(x_ref, w_ref, degp_ref, q_ref, dinv_ref):
    deg = degp_ref[0] + degp_ref[1] + 1.0
    dinv = lax.rsqrt(deg)
    p = jnp.dot(x_ref[...], w_ref[...], preferred_element_type=jnp.float32)
    dinv_ref[...] = dinv
    q_ref[...] = dinv * p


def _tc2_body(s1p_ref, q1_ref, dinv_ref, b1_ref, q2_ref):
    dinv = dinv_ref[...]
    agg = dinv * (s1p_ref[0] + s1p_ref[1] + q1_ref[...])
    h = jnp.maximum(agg + b1_ref[...], 0.0)
    q2_ref[...] = dinv * h


def _tc3_body(s2p_ref, q2_ref, dinv_ref, w2_ref, b2_ref, o_ref):
    agg = dinv_ref[...] * (s2p_ref[0] + s2p_ref[1] + q2_ref[...])
    z = jnp.dot(agg, w2_ref[...], preferred_element_type=jnp.float32)
    z = z + b2_ref[...]
    z = z - jnp.max(z, axis=1, keepdims=True)
    o_ref[...] = z - jnp.log(jnp.sum(jnp.exp(z), axis=1, keepdims=True))


def _row_spec(w):
    return pl.BlockSpec((_ROW_BLK, w), lambda i: (i, 0))


def _part_spec(w):
    return pl.BlockSpec((NC, _ROW_BLK, w), lambda i: (0, i, 0))


def _full_spec(a, b):
    return pl.BlockSpec((a, b), lambda i: (0, 0))


_tc1 = pl.pallas_call(
    _tc1_body,
    grid=(_N_BLK,),
    in_specs=[_row_spec(D_IN), _full_spec(D_IN, D_HID), _part_spec(L)],
    out_specs=(_row_spec(D_HID), _row_spec(L)),
    out_shape=(jax.ShapeDtypeStruct((NP, D_HID), jnp.float32),
               jax.ShapeDtypeStruct((NP, L), jnp.float32)),
)

_tc2 = pl.pallas_call(
    _tc2_body,
    grid=(_N_BLK,),
    in_specs=[_part_spec(L), _row_spec(D_HID), _row_spec(L),
              _full_spec(1, D_HID)],
    out_specs=_row_spec(D_HID),
    out_shape=jax.ShapeDtypeStruct((NP, D_HID), jnp.float32),
)

_tc3 = pl.pallas_call(
    _tc3_body,
    grid=(_N_BLK,),
    in_specs=[_part_spec(L), _row_spec(D_HID), _row_spec(L),
              _full_spec(D_HID, N_CLASSES), _full_spec(1, N_CLASSES)],
    out_specs=_row_spec(N_CLASSES),
    out_shape=jax.ShapeDtypeStruct((NP, N_CLASSES), jnp.float32),
)


def kernel(x, edge_index, W1, b1, W2, b2):
    src_h = edge_index[0].reshape(NW, NCH, CH)
    dst_h = edge_index[1].reshape(NW, NCH, CH)
    x_pad = jnp.pad(x, ((0, NP - N), (0, 0)))

    degp = _deg_kernel(dst_h)
    q1, dinv = _tc1(x_pad, W1, degp)
    s1p = _agg_kernel(q1, src_h, dst_h)
    q2 = _tc2(s1p, q1, dinv, b1.reshape(1, D_HID))
    s2p = _agg_kernel(q2, src_h, dst_h)
    out = _tc3(s2p, q2, dinv, W2, b2.reshape(1, N_CLASSES))
    return out[:N]


# speedup vs baseline: 58.3080x; 1.2796x over previous
import functools

import jax
import jax.numpy as jnp
from jax import lax
from jax.experimental import pallas as pl
from jax.experimental.pallas import tpu as pltpu
from jax.experimental.pallas import tpu_sc as plsc

N = 10000
E = 320000
D_IN = 128
D_HID = 16
N_CLASSES = 40

NC, NS, L = 2, 16, 16
NW = NC * NS
EPT = E // NW
CH = 80
NCH = EPT // CH

NP = 10240
RPT = NP // NS

_ROW_BLK = 1280
_N_BLK = NP // _ROW_BLK

_MESH = plsc.VectorSubcoreMesh(core_axis_name="c", subcore_axis_name="s")
_SC_PARAMS = pltpu.CompilerParams(use_tc_tiling_on_sc=False)


@functools.partial(
    pl.kernel,
    out_type=jax.ShapeDtypeStruct((NC, NP, L), jnp.float32),
    mesh=_MESH,
    scratch_types=[
        pltpu.VMEM((NCH, CH), jnp.int32),
        pltpu.VMEM((CH, L), jnp.float32),
        pltpu.VMEM((RPT, L), jnp.float32),
        pltpu.VMEM_SHARED((NP, L), jnp.float32),
        pltpu.SemaphoreType.DMA,
        pltpu.SemaphoreType.DMA,
        pltpu.SemaphoreType.DMA,
        pltpu.SemaphoreType.DMA,
    ],
    compiler_params=_SC_PARAMS,
)
def _deg_kernel(dst_hbm, out_hbm, dst_v, ones_v, zero_v, acc_sh,
                ss0, ss1, ss2, ss3):
    cid = lax.axis_index("c")
    sid = lax.axis_index("s")
    wid = cid * NS + sid
    ssems = (ss0, ss1, ss2, ss3)
    pltpu.sync_copy(dst_hbm.at[wid], dst_v)

    def fill_ones(i, _):
        ones_v[i, :] = jnp.ones((L,), jnp.float32)
        return 0
    lax.fori_loop(0, CH, fill_ones, 0)

    def fill_zero(i, _):
        zero_v[i, :] = jnp.zeros((L,), jnp.float32)
        return 0
    lax.fori_loop(0, RPT, fill_zero, 0)
    pltpu.sync_copy(zero_v, acc_sh.at[pl.ds(sid * RPT, RPT)])
    plsc.subcore_barrier()

    @pl.loop(0, NCH - 1, step=4)
    def _(j):
        for k in range(4):
            idx = j + k

            @pl.when(idx >= 4)
            def _():
                pltpu.make_async_copy(ones_v, acc_sh.at[dst_v.at[0]],
                                      ssems[k]).wait()
            pltpu.async_copy(ones_v, acc_sh.at[dst_v.at[idx]], ssems[k],
                             add=True)
    pltpu.make_async_copy(ones_v, acc_sh.at[dst_v.at[0]], ssems[0]).wait()
    pltpu.async_copy(ones_v, acc_sh.at[dst_v.at[NCH - 1]], ssems[0], add=True)
    for k in range(4):
        pltpu.make_async_copy(ones_v, acc_sh.at[dst_v.at[0]], ssems[k]).wait()
    plsc.subcore_barrier()
    pltpu.sync_copy(acc_sh.at[pl.ds(sid * RPT, RPT)],
                    out_hbm.at[cid, pl.ds(sid * RPT, RPT)])


@functools.partial(
    pl.kernel,
    out_type=jax.ShapeDtypeStruct((NC, NP, L), jnp.float32),
    mesh=_MESH,
    scratch_types=[
        pltpu.VMEM((NCH, CH), jnp.int32),
        pltpu.VMEM((NCH, CH), jnp.int32),
        pltpu.VMEM((4, CH, L), jnp.float32),
        pltpu.VMEM((RPT, L), jnp.float32),
        pltpu.VMEM_SHARED((NP, L), jnp.float32),
        pltpu.VMEM_SHARED((NP, L), jnp.float32),
        pltpu.SemaphoreType.DMA,
        pltpu.SemaphoreType.DMA,
        pltpu.SemaphoreType.DMA,
        pltpu.SemaphoreType.DMA,
        pltpu.SemaphoreType.DMA,
        pltpu.SemaphoreType.DMA,
        pltpu.SemaphoreType.DMA,
        pltpu.SemaphoreType.DMA,
    ],
    compiler_params=_SC_PARAMS,
)
def _agg_kernel(q_hbm, src_hbm, dst_hbm, out_hbm,
                src_v, dst_v, buf, zero_v, q_sh, s_sh,
                gs0, gs1, gs2, gs3, ss0, ss1, ss2, ss3):
    cid = lax.axis_index("c")
    sid = lax.axis_index("s")
    wid = cid * NS + sid
    gsems = (gs0, gs1, gs2, gs3)
    ssems = (ss0, ss1, ss2, ss3)
    pltpu.sync_copy(src_hbm.at[wid], src_v)
    pltpu.sync_copy(dst_hbm.at[wid], dst_v)
    pltpu.sync_copy(q_hbm.at[pl.ds(sid * RPT, RPT)],
                    q_sh.at[pl.ds(sid * RPT, RPT)])

    def fill_zero(i, _):
        zero_v[i, :] = jnp.zeros((L,), jnp.float32)
        return 0
    lax.fori_loop(0, RPT, fill_zero, 0)
    pltpu.sync_copy(zero_v, s_sh.at[pl.ds(sid * RPT, RPT)])
    plsc.subcore_barrier()

    def gather(idx, k):
        pltpu.async_copy(q_sh.at[src_v.at[idx]], buf.at[k], gsems[k])

    def wait_gather(k):
        pltpu.make_async_copy(q_sh.at[src_v.at[0]], buf.at[k],
                              gsems[k]).wait()

    def scatter(idx, k):
        pltpu.async_copy(buf.at[k], s_sh.at[dst_v.at[idx]], ssems[k],
                         add=True)

    def wait_scatter(k):
        pltpu.make_async_copy(buf.at[k], s_sh.at[dst_v.at[0]],
                              ssems[k]).wait()

    gather(0, 0)
    gather(1, 1)

    @pl.loop(0, NCH - 1, step=4)
    def _(j):
        for k in range(4):
            idx = j + k
            wait_gather(k)
            scatter(idx, k)
            k2 = (k + 2) % 4

            @pl.when(idx + 2 < NCH)
            def _():
                @pl.when(idx >= 2)
                def _():
                    wait_scatter(k2)
                gather(idx + 2, k2)
    wait_gather(0)
    scatter(NCH - 1, 0)
    for k in (0, 2, 3):
        wait_scatter(k)
    plsc.subcore_barrier()
    pltpu.sync_copy(s_sh.at[pl.ds(sid * RPT, RPT)],
                    out_hbm.at[cid, pl.ds(sid * RPT, RPT)])


def _tc1_body(x_ref, w_ref, degp_ref, q_ref, dinv_ref):
    deg = degp_ref[0] + degp_ref[1] + 1.0
    dinv = lax.rsqrt(deg)
    p = jnp.dot(x_ref[...], w_ref[...], preferred_element_type=jnp.float32)
    dinv_ref[...] = dinv
    q_ref[...] = dinv * p


def _tc2_body(s1p_ref, q1_ref, dinv_ref, b1_ref, q2_ref):
    dinv = dinv_ref[...]
    agg = dinv * (s1p_ref[0] + s1p_ref[1] + q1_ref[...])
    h = jnp.maximum(agg + b1_ref[...], 0.0)
    q2_ref[...] = dinv * h


def _tc3_body(s2p_ref, q2_ref, dinv_ref, w2_ref, b2_ref, o_ref):
    agg = dinv_ref[...] * (s2p_ref[0] + s2p_ref[1] + q2_ref[...])
    z = jnp.dot(agg, w2_ref[...], preferred_element_type=jnp.float32)
    z = z + b2_ref[...]
    z = z - jnp.max(z, axis=1, keepdims=True)
    o_ref[...] = z - jnp.log(jnp.sum(jnp.exp(z), axis=1, keepdims=True))


def _row_spec(w):
    return pl.BlockSpec((_ROW_BLK, w), lambda i: (i, 0))


def _part_spec(w):
    return pl.BlockSpec((NC, _ROW_BLK, w), lambda i: (0, i, 0))


def _full_spec(a, b):
    return pl.BlockSpec((a, b), lambda i: (0, 0))


_tc1 = pl.pallas_call(
    _tc1_body,
    grid=(_N_BLK,),
    in_specs=[_row_spec(D_IN), _full_spec(D_IN, D_HID), _part_spec(L)],
    out_specs=(_row_spec(D_HID), _row_spec(L)),
    out_shape=(jax.ShapeDtypeStruct((NP, D_HID), jnp.float32),
               jax.ShapeDtypeStruct((NP, L), jnp.float32)),
)

_tc2 = pl.pallas_call(
    _tc2_body,
    grid=(_N_BLK,),
    in_specs=[_part_spec(L), _row_spec(D_HID), _row_spec(L),
              _full_spec(1, D_HID)],
    out_specs=_row_spec(D_HID),
    out_shape=jax.ShapeDtypeStruct((NP, D_HID), jnp.float32),
)

_tc3 = pl.pallas_call(
    _tc3_body,
    grid=(_N_BLK,),
    in_specs=[_part_spec(L), _row_spec(D_HID), _row_spec(L),
              _full_spec(D_HID, N_CLASSES), _full_spec(1, N_CLASSES)],
    out_specs=_row_spec(N_CLASSES),
    out_shape=jax.ShapeDtypeStruct((NP, N_CLASSES), jnp.float32),
)


def kernel(x, edge_index, W1, b1, W2, b2):
    src_h = edge_index[0].reshape(NW, NCH, CH)
    dst_h = edge_index[1].reshape(NW, NCH, CH)
    x_pad = jnp.pad(x, ((0, NP - N), (0, 0)))

    degp = _deg_kernel(dst_h)
    q1, dinv = _tc1(x_pad, W1, degp)
    s1p = _agg_kernel(q1, src_h, dst_h)
    q2 = _tc2(s1p, q1, dinv, b1.reshape(1, D_HID))
    s2p = _agg_kernel(q2, src_h, dst_h)
    out = _tc3(s2p, q2, dinv, W2, b2.reshape(1, N_CLASSES))
    return out[:N]
